# Initial kernel scaffold; baseline (speedup 1.0000x reference)
import jax
import jax.numpy as jnp
from jax.experimental import pallas as pl


def kernel(face_centers, face_normals, pose_extended, pos_table, normal_table, pose_table, W0, W1, W2):
    raise NotImplementedError("write your pallas kernel here")


# speedup vs baseline: 21.3712x; 21.3712x over previous
import functools

import numpy as np
import jax
import jax.numpy as jnp
from jax import lax
from jax.experimental import pallas as pl
from jax.experimental.pallas import tpu as pltpu
from jax.experimental.pallas import tpu_sc as plsc

_L = 16
_F = 2
_T = 1 << 19
_MASK = _T - 1
_N = 131072
_HID = 64
_ODIM = 9
_ENC = 3 * _L * _F

_NC = 2
_NS = 16
_NW = _NC * _NS
_PPT = _N // _NW
_CH = 128
_NCHUNK = _PPT // _CH
_NG = _CH // 16
_NSLOT = 32

_RES_F32 = np.floor(16.0 * (1.4472692012786865 ** np.arange(_L))).astype(np.float32)

_P1 = np.int32(np.int64(2654435761) - (1 << 32))
_P2 = np.int32(805459861)
_P3 = np.int32(np.int64(3674653429) - (1 << 32))

_mesh = plsc.VectorSubcoreMesh(core_axis_name="c", subcore_axis_name="s")


@functools.partial(
    pl.kernel,
    out_type=jax.ShapeDtypeStruct((_N, _ENC), jnp.float32),
    mesh=_mesh,
    compiler_params=pltpu.CompilerParams(needs_layout_passes=False,
                                         use_tc_tiling_on_sc=False),
    scratch_types=[
        pltpu.VMEM((16,), jnp.float32),
        pltpu.VMEM((10, _CH), jnp.float32),
        pltpu.VMEM((10, _CH), jnp.float32),
        pltpu.VMEM((_NSLOT, _CH), jnp.int32),
        pltpu.VMEM((_NSLOT, _CH), jnp.int32),
        pltpu.VMEM((_NSLOT * _CH, 16), jnp.float32),
        pltpu.VMEM((_CH, _ENC), jnp.float32),
        pltpu.VMEM((16,), jnp.float32),
        pltpu.SemaphoreType.DMA,
    ],
)
def _encode_sc(coords, pos_t, nrm_t, pose_t, res_in, out,
               res_v, inp_v, frac_v, idx_v, idxlo_v, rows_v, enc_v, w_v, sem):
    cid = lax.axis_index("c")
    sid = lax.axis_index("s")
    wid = sid * _NC + cid
    pltpu.sync_copy(res_in, res_v)

    lane = lax.iota(jnp.int32, 16)
    half = lane >> 1
    parity = lane & 1

    tables = ((0, 3, 0, 0), (3, 3, 8, 32), (6, 4, 16, 64))

    def chunk_body(ch, carry):
        pltpu.sync_copy(coords.at[wid, ch], inp_v)

        def level_body(l, carry2):
            resb = plsc.load_gather(res_v, [jnp.full((16,), l, jnp.int32)])
            lofs = l * _T

            def idx_body(g, c3):
                gof = g * 16
                for roff, nd, soff, _coff in tables:
                    h0 = []
                    h1 = []
                    for dd in range(nd):
                        x = inp_v[roff + dd, pl.ds(gof, 16)]
                        pos = x * resb
                        ip = pos.astype(jnp.int32)
                        frac_v[roff + dd, pl.ds(gof, 16)] = pos - ip.astype(jnp.float32)
                        if dd == 0:
                            a0 = ip
                            a1 = ip + 1
                        else:
                            p = (_P1, _P2, _P3)[dd - 1]
                            a0 = ip * p
                            a1 = a0 + p
                        h0.append(a0)
                        h1.append(a1)
                    for c in range(1 << nd):
                        h = None
                        for dd in range(nd):
                            term = h1[dd] if (c >> (nd - 1 - dd)) & 1 else h0[dd]
                            h = term if h is None else h ^ term
                        full = (h & _MASK) + lofs
                        idx_v[soff + c, pl.ds(gof, 16)] = full >> 3
                        idxlo_v[soff + c, pl.ds(gof, 16)] = (full & 7) * 2
                return c3

            lax.fori_loop(0, _NG, idx_body, 0)

            copies = []
            for s in range(_NSLOT):
                tref = pos_t if s < 8 else (nrm_t if s < 16 else pose_t)
                copies.append(pltpu.async_copy(
                    tref.at[idx_v.at[s]],
                    rows_v.at[pl.ds(s * _CH, _CH)],
                    sem))
            for cp in copies:
                cp.wait()

            colbase = 2 * l + parity

            def acc_body(g, c3):
                gof = g * 16
                row_a = gof + half
                row_b = row_a + 8
                for roff, nd, soff, coff in tables:
                    fr = [frac_v[roff + dd, pl.ds(gof, 16)] for dd in range(nd)]
                    om = [1.0 - f for f in fr]
                    p01 = [om[0] * om[1], om[0] * fr[1], fr[0] * om[1], fr[0] * fr[1]]
                    if nd == 3:
                        tail = [om[2], fr[2]]
                    else:
                        tail = [om[2] * om[3], om[2] * fr[3], fr[2] * om[3], fr[2] * fr[3]]
                    facc0 = None
                    facc1 = None
                    for c in range(1 << nd):
                        w = p01[c >> (nd - 2)] * tail[c & ((1 << (nd - 2)) - 1)]
                        w_v[...] = w
                        w_a = plsc.load_gather(w_v, [half])
                        w_b = plsc.load_gather(w_v, [half + 8])
                        rbase = (soff + c) * _CH
                        slot = jnp.full((16,), soff + c, jnp.int32)
                        col_a = plsc.load_gather(idxlo_v, [slot, row_a]) + parity
                        col_b = plsc.load_gather(idxlo_v, [slot, row_b]) + parity
                        v_a = plsc.load_gather(rows_v, [rbase + row_a, col_a])
                        v_b = plsc.load_gather(rows_v, [rbase + row_b, col_b])
                        if facc0 is None:
                            facc0 = w_a * v_a
                            facc1 = w_b * v_b
                        else:
                            facc0 = facc0 + w_a * v_a
                            facc1 = facc1 + w_b * v_b
                    colv = colbase + coff
                    plsc.store_scatter(enc_v, [gof + half, colv], facc0)
                    plsc.store_scatter(enc_v, [gof + 8 + half, colv], facc1)
                return c3

            lax.fori_loop(0, _NG, acc_body, 0)
            return carry2

        lax.fori_loop(0, _L, level_body, 0)
        pltpu.sync_copy(enc_v, out.at[pl.ds(wid * _PPT + ch * _CH, _CH)])
        return carry

    lax.fori_loop(0, _NCHUNK, chunk_body, 0)


_BN = 1024


def _mlp_body(enc_ref, w0_ref, w1_ref, w2_ref, out_ref):
    h = jnp.maximum(
        jnp.dot(enc_ref[...], w0_ref[...], preferred_element_type=jnp.float32,
                precision=lax.Precision.HIGHEST), 0.0)
    h = jnp.maximum(
        jnp.dot(h, w1_ref[...], preferred_element_type=jnp.float32,
                precision=lax.Precision.HIGHEST), 0.0)
    out_ref[...] = jnp.dot(h, w2_ref[...], preferred_element_type=jnp.float32,
                           precision=lax.Precision.HIGHEST)


def _mlp(enc, W0, W1, W2):
    return pl.pallas_call(
        _mlp_body,
        grid=(_N // _BN,),
        in_specs=[
            pl.BlockSpec((_BN, _ENC), lambda i: (i, 0)),
            pl.BlockSpec((_ENC, _HID), lambda i: (0, 0)),
            pl.BlockSpec((_HID, _HID), lambda i: (0, 0)),
            pl.BlockSpec((_HID, _ODIM), lambda i: (0, 0)),
        ],
        out_specs=pl.BlockSpec((_BN, _ODIM), lambda i: (i, 0)),
        out_shape=jax.ShapeDtypeStruct((_N, _ODIM), jnp.float32),
    )(enc, W0, W1, W2)


def kernel(face_centers, face_normals, pose_extended, pos_table, normal_table,
           pose_table, W0, W1, W2):
    coords = jnp.concatenate(
        [face_centers.T, face_normals.T, pose_extended.T], axis=0)
    coords = coords.reshape(10, _NW, _NCHUNK, _CH).transpose(1, 2, 0, 3)
    enc = _encode_sc(coords,
                     pos_table.reshape(_L * _T * _F // 16, 16),
                     normal_table.reshape(_L * _T * _F // 16, 16),
                     pose_table.reshape(_L * _T * _F // 16, 16),
                     jnp.asarray(_RES_F32))
    return _mlp(enc, W0, W1, W2)


# speedup vs baseline: 85.7541x; 4.0126x over previous
import functools

import numpy as np
import jax
import jax.numpy as jnp
from jax import lax
from jax.experimental import pallas as pl
from jax.experimental.pallas import tpu as pltpu
from jax.experimental.pallas import tpu_sc as plsc

_L = 16
_F = 2
_T = 1 << 19
_MASK = _T - 1
_N = 131072
_HID = 64
_ODIM = 9
_ENC = 3 * _L * _F

_NC = 2
_NS = 16
_NW = _NC * _NS
_PPT = _N // _NW
_CH = 64
_NCHUNK = _PPT // _CH
_NG = _CH // 16
_NSLOT = 32

_RES_F32 = np.floor(16.0 * (1.4472692012786865 ** np.arange(_L))).astype(np.float32)

_P1 = np.int32(np.int64(2654435761) - (1 << 32))
_P2 = np.int32(805459861)
_P3 = np.int32(np.int64(3674653429) - (1 << 32))

_mesh = plsc.VectorSubcoreMesh(core_axis_name="c", subcore_axis_name="s")


@functools.partial(
    pl.kernel,
    out_type=jax.ShapeDtypeStruct((_N, _ENC), jnp.float32),
    mesh=_mesh,
    compiler_params=pltpu.CompilerParams(needs_layout_passes=False,
                                         use_tc_tiling_on_sc=False),
    scratch_types=[
        pltpu.VMEM((16,), jnp.float32),
        pltpu.VMEM((10, _CH), jnp.float32),
        pltpu.VMEM((10, _CH), jnp.float32),
        pltpu.VMEM((_NSLOT, 2 * _CH), jnp.int32),
        pltpu.VMEM((_NSLOT, _CH), jnp.int32),
        pltpu.VMEM((_NSLOT * 2 * _CH, 16), jnp.float32),
        pltpu.VMEM((_CH, _ENC), jnp.float32),
        pltpu.VMEM((16,), jnp.float32),
        pltpu.SemaphoreType.DMA,
    ],
)
def _encode_sc(coords, pos_t, nrm_t, pose_t, res_in, out,
               res_v, inp_v, frac_v, idx_v, idxlo_v, rows_v, enc_v, w_v, sem):
    cid = lax.axis_index("c")
    sid = lax.axis_index("s")
    wid = sid * _NC + cid
    pltpu.sync_copy(res_in, res_v)

    lane = lax.iota(jnp.int32, 16)
    half = lane >> 1
    parity = lane & 1
    pf = parity * _CH

    tables = ((0, 3, 0, 0), (3, 3, 8, 32), (6, 4, 16, 64))

    def chunk_body(ch, carry):
        pltpu.sync_copy(coords.at[wid, ch], inp_v)

        def level_body(l, carry2):
            resb = plsc.load_gather(res_v, [jnp.full((16,), l, jnp.int32)])
            lofs = l * (_T * _F // 16)

            def idx_body(g, c3):
                gof = g * 16
                for roff, nd, soff, _coff in tables:
                    h0 = []
                    h1 = []
                    for dd in range(nd):
                        x = inp_v[roff + dd, pl.ds(gof, 16)]
                        pos = x * resb
                        ip = pos.astype(jnp.int32)
                        frac_v[roff + dd, pl.ds(gof, 16)] = pos - ip.astype(jnp.float32)
                        if dd == 0:
                            a0 = ip
                            a1 = ip + 1
                        else:
                            p = (_P1, _P2, _P3)[dd - 1]
                            a0 = ip * p
                            a1 = a0 + p
                        h0.append(a0)
                        h1.append(a1)
                    for c in range(1 << nd):
                        h = None
                        for dd in range(nd):
                            term = h1[dd] if (c >> (nd - 1 - dd)) & 1 else h0[dd]
                            h = term if h is None else h ^ term
                        t = h & _MASK
                        r0 = (t >> 4) + ((t >> 7) << 3) + lofs
                        idx_v[soff + c, pl.ds(gof, 16)] = r0
                        idx_v[soff + c, pl.ds(_CH + gof, 16)] = r0 + 8
                        idxlo_v[soff + c, pl.ds(gof, 16)] = t & 15
                return c3

            lax.fori_loop(0, _NG, idx_body, 0)

            copies = []
            for s in range(_NSLOT):
                tref = pos_t if s < 8 else (nrm_t if s < 16 else pose_t)
                copies.append(pltpu.async_copy(
                    tref.at[idx_v.at[s]],
                    rows_v.at[pl.ds(s * 2 * _CH, 2 * _CH)],
                    sem))
            for cp in copies:
                cp.wait()

            colbase = 2 * l + parity

            def acc_body(g, c3):
                gof = g * 16
                row_a = gof + half
                row_b = row_a + 8
                for roff, nd, soff, coff in tables:
                    fr = [frac_v[roff + dd, pl.ds(gof, 16)] for dd in range(nd)]
                    om = [1.0 - f for f in fr]
                    p01 = [om[0] * om[1], om[0] * fr[1], fr[0] * om[1], fr[0] * fr[1]]
                    if nd == 3:
                        tail = [om[2], fr[2]]
                    else:
                        tail = [om[2] * om[3], om[2] * fr[3], fr[2] * om[3], fr[2] * fr[3]]
                    facc0 = None
                    facc1 = None
                    for c in range(1 << nd):
                        w = p01[c >> (nd - 2)] * tail[c & ((1 << (nd - 2)) - 1)]
                        w_v[...] = w
                        w_a = plsc.load_gather(w_v, [half])
                        w_b = plsc.load_gather(w_v, [half + 8])
                        slot = jnp.full((16,), soff + c, jnp.int32)
                        col_a = plsc.load_gather(idxlo_v, [slot, row_a])
                        col_b = plsc.load_gather(idxlo_v, [slot, row_b])
                        rbase = (soff + c) * 2 * _CH
                        v_a = plsc.load_gather(rows_v, [rbase + pf + row_a, col_a])
                        v_b = plsc.load_gather(rows_v, [rbase + pf + row_b, col_b])
                        if facc0 is None:
                            facc0 = w_a * v_a
                            facc1 = w_b * v_b
                        else:
                            facc0 = facc0 + w_a * v_a
                            facc1 = facc1 + w_b * v_b
                    colv = colbase + coff
                    plsc.store_scatter(enc_v, [gof + half, colv], facc0)
                    plsc.store_scatter(enc_v, [gof + 8 + half, colv], facc1)
                return c3

            lax.fori_loop(0, _NG, acc_body, 0)
            return carry2

        lax.fori_loop(0, _L, level_body, 0)
        pltpu.sync_copy(enc_v, out.at[pl.ds(wid * _PPT + ch * _CH, _CH)])
        return carry

    lax.fori_loop(0, _NCHUNK, chunk_body, 0)


_BN = 1024


def _mlp_body(enc_ref, w0_ref, w1_ref, w2_ref, out_ref):
    h = jnp.maximum(
        jnp.dot(enc_ref[...], w0_ref[...], preferred_element_type=jnp.float32,
                precision=lax.Precision.HIGHEST), 0.0)
    h = jnp.maximum(
        jnp.dot(h, w1_ref[...], preferred_element_type=jnp.float32,
                precision=lax.Precision.HIGHEST), 0.0)
    out_ref[...] = jnp.dot(h, w2_ref[...], preferred_element_type=jnp.float32,
                           precision=lax.Precision.HIGHEST)


def _mlp(enc, W0, W1, W2):
    return pl.pallas_call(
        _mlp_body,
        grid=(_N // _BN,),
        in_specs=[
            pl.BlockSpec((_BN, _ENC), lambda i: (i, 0)),
            pl.BlockSpec((_ENC, _HID), lambda i: (0, 0)),
            pl.BlockSpec((_HID, _HID), lambda i: (0, 0)),
            pl.BlockSpec((_HID, _ODIM), lambda i: (0, 0)),
        ],
        out_specs=pl.BlockSpec((_BN, _ODIM), lambda i: (i, 0)),
        out_shape=jax.ShapeDtypeStruct((_N, _ODIM), jnp.float32),
    )(enc, W0, W1, W2)


def kernel(face_centers, face_normals, pose_extended, pos_table, normal_table,
           pose_table, W0, W1, W2):
    coords = jnp.concatenate(
        [face_centers.T, face_normals.T, pose_extended.T], axis=0)
    coords = coords.reshape(10, _NW, _NCHUNK, _CH).transpose(1, 2, 0, 3)

    def native_view(tbl):
        return tbl.reshape(_L, _T // 128, 128, _F).transpose(0, 1, 3, 2).reshape(
            _L * _T * _F // 16, 16)

    enc = _encode_sc(coords,
                     native_view(pos_table),
                     native_view(normal_table),
                     native_view(pose_table),
                     jnp.asarray(_RES_F32))
    return _mlp(enc, W0, W1, W2)


# speedup vs baseline: 123.7018x; 1.4425x over previous
import functools

import numpy as np
import jax
import jax.numpy as jnp
from jax import lax
from jax.experimental import pallas as pl
from jax.experimental.pallas import tpu as pltpu
from jax.experimental.pallas import tpu_sc as plsc

_L = 16
_F = 2
_T = 1 << 19
_MASK = _T - 1
_N = 131072
_HID = 64
_ODIM = 9
_ENC = 3 * _L * _F
_TROWS = _L * _T * _F // 16

_NC = 2
_NS = 16
_NW = _NC * _NS
_PPT = _N // _NW
_CH = 128
_NCHUNK = _PPT // _CH
_NG = _CH // 16
_NSLOT = 32

_CBLK = 64
_BPT = _TROWS * 16 // 256 // _NW

_RES_F32 = np.floor(16.0 * (1.4472692012786865 ** np.arange(_L))).astype(np.float32)

_P1 = np.int32(np.int64(2654435761) - (1 << 32))
_P2 = np.int32(805459861)
_P3 = np.int32(np.int64(3674653429) - (1 << 32))

_mesh = plsc.VectorSubcoreMesh(core_axis_name="c", subcore_axis_name="s")
_sc_params = pltpu.CompilerParams(needs_layout_passes=False,
                                  use_tc_tiling_on_sc=False)

_tbl_sds = jax.ShapeDtypeStruct((_TROWS, 16), jnp.float32)


@functools.partial(
    pl.kernel,
    out_type=(_tbl_sds, _tbl_sds, _tbl_sds),
    mesh=_mesh,
    compiler_params=_sc_params,
    scratch_types=[
        pltpu.VMEM((_CBLK * 16, 16), jnp.float32),
        pltpu.VMEM((_CBLK * 16, 16), jnp.float32),
    ],
)
def _convert_tables(t0, t1, t2, o0, o1, o2, in_v, out_v):
    cid = lax.axis_index("c")
    sid = lax.axis_index("s")
    wid = sid * _NC + cid
    lane = lax.iota(jnp.int32, 16)
    half = lane >> 1
    parity = lane & 1
    p8 = parity * 8

    for tin, tout in ((t0, o0), (t1, o1), (t2, o2)):
        def chunk_body(cix, carry):
            row0 = (wid * _BPT + cix * _CBLK) * 16
            pltpu.sync_copy(tin.at[pl.ds(row0, _CBLK * 16)], in_v)

            def blk_body(b, c2):
                for k in range(16):
                    src = plsc.load_gather(
                        in_v, [b * 16 + (k >> 1) + p8, 8 * (k & 1) + half])
                    out_v[b * 16 + k, :] = src
                return c2

            lax.fori_loop(0, _CBLK, blk_body, 0)
            pltpu.sync_copy(out_v, tout.at[pl.ds(row0, _CBLK * 16)])
            return carry

        lax.fori_loop(0, _BPT // _CBLK, chunk_body, 0)


@functools.partial(
    pl.kernel,
    out_type=jax.ShapeDtypeStruct((_N, _ENC), jnp.float32),
    mesh=_mesh,
    compiler_params=_sc_params,
    scratch_types=[
        pltpu.VMEM((16,), jnp.float32),
        pltpu.VMEM((10, _CH), jnp.float32),
        pltpu.VMEM((10, _CH), jnp.float32),
        pltpu.VMEM((_NSLOT, _CH), jnp.int32),
        pltpu.VMEM((_NSLOT, _CH), jnp.int32),
        pltpu.VMEM((_NSLOT * _CH, 16), jnp.float32),
        pltpu.VMEM((_CH, _ENC), jnp.float32),
        pltpu.VMEM((16,), jnp.float32),
        pltpu.SemaphoreType.DMA,
    ],
)
def _encode_sc(coords, pos_t, nrm_t, pose_t, res_in, out,
               res_v, inp_v, frac_v, idx_v, idxlo_v, rows_v, enc_v, w_v, sem):
    cid = lax.axis_index("c")
    sid = lax.axis_index("s")
    wid = sid * _NC + cid
    pltpu.sync_copy(res_in, res_v)

    lane = lax.iota(jnp.int32, 16)
    half = lane >> 1
    parity = lane & 1

    tables = ((0, 3, 0, 0), (3, 3, 8, 32), (6, 4, 16, 64))

    def chunk_body(ch, carry):
        pltpu.sync_copy(coords.at[wid, ch], inp_v)

        def level_body(l, carry2):
            resb = plsc.load_gather(res_v, [jnp.full((16,), l, jnp.int32)])
            lofs = l * (_T * _F // 16)

            def idx_body(g, c3):
                gof = g * 16
                for roff, nd, soff, _coff in tables:
                    h0 = []
                    h1 = []
                    for dd in range(nd):
                        x = inp_v[roff + dd, pl.ds(gof, 16)]
                        pos = x * resb
                        ip = pos.astype(jnp.int32)
                        frac_v[roff + dd, pl.ds(gof, 16)] = pos - ip.astype(jnp.float32)
                        if dd == 0:
                            a0 = ip
                            a1 = ip + 1
                        else:
                            p = (_P1, _P2, _P3)[dd - 1]
                            a0 = ip * p
                            a1 = a0 + p
                        h0.append(a0)
                        h1.append(a1)
                    for c in range(1 << nd):
                        h = None
                        for dd in range(nd):
                            term = h1[dd] if (c >> (nd - 1 - dd)) & 1 else h0[dd]
                            h = term if h is None else h ^ term
                        t = h & _MASK
                        idx_v[soff + c, pl.ds(gof, 16)] = (t >> 3) + lofs
                        idxlo_v[soff + c, pl.ds(gof, 16)] = (t & 7) * 2
                return c3

            lax.fori_loop(0, _NG, idx_body, 0)

            copies = []
            for s in range(_NSLOT):
                tref = pos_t if s < 8 else (nrm_t if s < 16 else pose_t)
                copies.append(pltpu.async_copy(
                    tref.at[idx_v.at[s]],
                    rows_v.at[pl.ds(s * _CH, _CH)],
                    sem))
            for cp in copies:
                cp.wait()

            colbase = 2 * l + parity

            def acc_body(g, c3):
                gof = g * 16
                row_a = gof + half
                row_b = row_a + 8
                for roff, nd, soff, coff in tables:
                    fr = [frac_v[roff + dd, pl.ds(gof, 16)] for dd in range(nd)]
                    om = [1.0 - f for f in fr]
                    p01 = [om[0] * om[1], om[0] * fr[1], fr[0] * om[1], fr[0] * fr[1]]
                    if nd == 3:
                        tail = [om[2], fr[2]]
                    else:
                        tail = [om[2] * om[3], om[2] * fr[3], fr[2] * om[3], fr[2] * fr[3]]
                    facc0 = None
                    facc1 = None
                    for c in range(1 << nd):
                        w = p01[c >> (nd - 2)] * tail[c & ((1 << (nd - 2)) - 1)]
                        w_v[...] = w
                        w_a = plsc.load_gather(w_v, [half])
                        w_b = plsc.load_gather(w_v, [half + 8])
                        slot = jnp.full((16,), soff + c, jnp.int32)
                        col_a = plsc.load_gather(idxlo_v, [slot, row_a]) + parity
                        col_b = plsc.load_gather(idxlo_v, [slot, row_b]) + parity
                        rbase = (soff + c) * _CH
                        v_a = plsc.load_gather(rows_v, [rbase + row_a, col_a])
                        v_b = plsc.load_gather(rows_v, [rbase + row_b, col_b])
                        if facc0 is None:
                            facc0 = w_a * v_a
                            facc1 = w_b * v_b
                        else:
                            facc0 = facc0 + w_a * v_a
                            facc1 = facc1 + w_b * v_b
                    colv = colbase + coff
                    plsc.store_scatter(enc_v, [gof + half, colv], facc0)
                    plsc.store_scatter(enc_v, [gof + 8 + half, colv], facc1)
                return c3

            lax.fori_loop(0, _NG, acc_body, 0)
            return carry2

        lax.fori_loop(0, _L, level_body, 0)
        pltpu.sync_copy(enc_v, out.at[pl.ds(wid * _PPT + ch * _CH, _CH)])
        return carry

    lax.fori_loop(0, _NCHUNK, chunk_body, 0)


_BN = 1024


def _mlp_body(enc_ref, w0_ref, w1_ref, w2_ref, out_ref):
    h = jnp.maximum(
        jnp.dot(enc_ref[...], w0_ref[...], preferred_element_type=jnp.float32,
                precision=lax.Precision.HIGHEST), 0.0)
    h = jnp.maximum(
        jnp.dot(h, w1_ref[...], preferred_element_type=jnp.float32,
                precision=lax.Precision.HIGHEST), 0.0)
    out_ref[...] = jnp.dot(h, w2_ref[...], preferred_element_type=jnp.float32,
                           precision=lax.Precision.HIGHEST)


def _mlp(enc, W0, W1, W2):
    return pl.pallas_call(
        _mlp_body,
        grid=(_N // _BN,),
        in_specs=[
            pl.BlockSpec((_BN, _ENC), lambda i: (i, 0)),
            pl.BlockSpec((_ENC, _HID), lambda i: (0, 0)),
            pl.BlockSpec((_HID, _HID), lambda i: (0, 0)),
            pl.BlockSpec((_HID, _ODIM), lambda i: (0, 0)),
        ],
        out_specs=pl.BlockSpec((_BN, _ODIM), lambda i: (i, 0)),
        out_shape=jax.ShapeDtypeStruct((_N, _ODIM), jnp.float32),
    )(enc, W0, W1, W2)


def kernel(face_centers, face_normals, pose_extended, pos_table, normal_table,
           pose_table, W0, W1, W2):
    coords = jnp.concatenate(
        [face_centers.T, face_normals.T, pose_extended.T], axis=0)
    coords = coords.reshape(10, _NW, _NCHUNK, _CH).transpose(1, 2, 0, 3)

    def native_view(tbl):
        return tbl.reshape(_L, _T // 128, 128, _F).transpose(0, 1, 3, 2).reshape(
            _TROWS, 16)

    pos_c, nrm_c, pose_c = _convert_tables(
        native_view(pos_table), native_view(normal_table),
        native_view(pose_table))
    enc = _encode_sc(coords, pos_c, nrm_c, pose_c, jnp.asarray(_RES_F32))
    return _mlp(enc, W0, W1, W2)


# speedup vs baseline: 201.2040x; 1.6265x over previous
import functools

import numpy as np
import jax
import jax.numpy as jnp
from jax import lax
from jax.experimental import pallas as pl
from jax.experimental.pallas import tpu as pltpu
from jax.experimental.pallas import tpu_sc as plsc

_L = 16
_F = 2
_T = 1 << 19
_MASK = _T - 1
_N = 131072
_HID = 64
_ODIM = 9
_ENC = 3 * _L * _F
_TROWS = _L * _T * _F // 16

_NC = 2
_NS = 16
_NW = _NC * _NS
_PPT = _N // _NW
_CH = 64
_NCHUNK = _PPT // _CH
_NG = _CH // 16
_NSLOT = 32
_NSTRM = _NSLOT // 2

_CBLK = 64
_BPT = _TROWS * 16 // 256 // _NW

_RES_F32 = np.floor(16.0 * (1.4472692012786865 ** np.arange(_L))).astype(np.float32)

_P1 = np.int32(np.int64(2654435761) - (1 << 32))
_P2 = np.int32(805459861)
_P3 = np.int32(np.int64(3674653429) - (1 << 32))

_mesh = plsc.VectorSubcoreMesh(core_axis_name="c", subcore_axis_name="s")
_sc_params = pltpu.CompilerParams(needs_layout_passes=False,
                                  use_tc_tiling_on_sc=False)

_tbl_sds = jax.ShapeDtypeStruct((_TROWS, 16), jnp.float32)


@functools.partial(
    pl.kernel,
    out_type=(_tbl_sds, _tbl_sds, _tbl_sds),
    mesh=_mesh,
    compiler_params=_sc_params,
    scratch_types=[
        pltpu.VMEM((_CBLK * 16, 16), jnp.float32),
        pltpu.VMEM((_CBLK * 16, 16), jnp.float32),
    ],
)
def _convert_tables(t0, t1, t2, o0, o1, o2, in_v, out_v):
    cid = lax.axis_index("c")
    sid = lax.axis_index("s")
    wid = sid * _NC + cid
    lane = lax.iota(jnp.int32, 16)
    half = lane >> 1
    parity = lane & 1
    p8 = parity * 8

    for tin, tout in ((t0, o0), (t1, o1), (t2, o2)):
        def chunk_body(cix, carry):
            row0 = (wid * _BPT + cix * _CBLK) * 16
            pltpu.sync_copy(tin.at[pl.ds(row0, _CBLK * 16)], in_v)

            def blk_body(b, c2):
                for k in range(16):
                    src = plsc.load_gather(
                        in_v, [b * 16 + (k >> 1) + p8, 8 * (k & 1) + half])
                    out_v[b * 16 + k, :] = src
                return c2

            lax.fori_loop(0, _CBLK, blk_body, 0)
            pltpu.sync_copy(out_v, tout.at[pl.ds(row0, _CBLK * 16)])
            return carry

        lax.fori_loop(0, _BPT // _CBLK, chunk_body, 0)


@functools.partial(
    pl.kernel,
    out_type=jax.ShapeDtypeStruct((_ENC, _N), jnp.float32),
    mesh=_mesh,
    compiler_params=_sc_params,
    scratch_types=[
        pltpu.VMEM((16,), jnp.float32),
        pltpu.VMEM((10, _CH), jnp.float32),
        pltpu.VMEM((2, 10, _CH), jnp.float32),
        pltpu.VMEM((2, _NSTRM, 2 * _CH), jnp.int32),
        pltpu.VMEM((2, _NSLOT, _CH), jnp.int32),
        pltpu.VMEM((2, _NSLOT * _CH, 16), jnp.float32),
        pltpu.VMEM((_ENC, _CH), jnp.float32),
        pltpu.SemaphoreType.DMA((2,)),
    ],
)
def _encode_sc(coords, pos_t, nrm_t, pose_t, res_in, out,
               res_v, inp_v, frac_v, idx_v, idxlo_v, rows_v, enc_v, sems):
    cid = lax.axis_index("c")
    sid = lax.axis_index("s")
    wid = sid * _NC + cid
    pltpu.sync_copy(res_in, res_v)

    lane = lax.iota(jnp.int32, 16)

    tables = ((0, 3, 0, 0), (3, 3, 8, 32), (6, 4, 16, 64))

    def idx_phase(l, buf):
        resb = plsc.load_gather(res_v, [jnp.full((16,), l, jnp.int32)])
        lofs = l * (_T * _F // 16)

        def idx_body(g, c3):
            gof = g * 16
            for roff, nd, soff, _coff in tables:
                h0 = []
                h1 = []
                for dd in range(nd):
                    x = inp_v[roff + dd, pl.ds(gof, 16)]
                    pos = x * resb
                    ip = pos.astype(jnp.int32)
                    frac_v[buf, roff + dd, pl.ds(gof, 16)] = pos - ip.astype(jnp.float32)
                    if dd == 0:
                        a0 = ip
                        a1 = ip + 1
                    else:
                        p = (_P1, _P2, _P3)[dd - 1]
                        a0 = ip * p
                        a1 = a0 + p
                    h0.append(a0)
                    h1.append(a1)
                for c in range(1 << nd):
                    h = None
                    for dd in range(nd):
                        term = h1[dd] if (c >> (nd - 1 - dd)) & 1 else h0[dd]
                        h = term if h is None else h ^ term
                    t = h & _MASK
                    s = soff + c
                    idx_v[buf, s >> 1, pl.ds((s & 1) * _CH + gof, 16)] = (t >> 3) + lofs
                    idxlo_v[buf, s, pl.ds(gof, 16)] = (t & 7) * 2
            return c3

        lax.fori_loop(0, _NG, idx_body, 0)

    def fire(buf, sem):
        for p in range(_NSTRM):
            tref = pos_t if p < 4 else (nrm_t if p < 8 else pose_t)
            pltpu.async_copy(
                tref.at[idx_v.at[buf, p]],
                rows_v.at[buf, pl.ds(p * 2 * _CH, 2 * _CH)],
                sem)

    def drain(buf, sem):
        pltpu.make_async_copy(
            pos_t.at[pl.ds(0, _NSLOT * _CH)],
            rows_v.at[buf],
            sem).wait()

    def acc_phase(l, buf):
        def acc_body(g, c3):
            gof = g * 16
            rowv = gof + lane
            for roff, nd, soff, coff in tables:
                fr = [frac_v[buf, roff + dd, pl.ds(gof, 16)] for dd in range(nd)]
                om = [1.0 - f for f in fr]
                p01 = [om[0] * om[1], om[0] * fr[1], fr[0] * om[1], fr[0] * fr[1]]
                if nd == 3:
                    tail = [om[2], fr[2]]
                else:
                    tail = [om[2] * om[3], om[2] * fr[3], fr[2] * om[3], fr[2] * fr[3]]
                facc0 = None
                facc1 = None
                for c in range(1 << nd):
                    w = p01[c >> (nd - 2)] * tail[c & ((1 << (nd - 2)) - 1)]
                    s = soff + c
                    col0 = idxlo_v[buf, s, pl.ds(gof, 16)]
                    v0 = plsc.load_gather(rows_v, [jnp.full((16,), buf, jnp.int32),
                                                   s * _CH + rowv, col0])
                    v1 = plsc.load_gather(rows_v, [jnp.full((16,), buf, jnp.int32),
                                                   s * _CH + rowv, col0 + 1])
                    if facc0 is None:
                        facc0 = w * v0
                        facc1 = w * v1
                    else:
                        facc0 = facc0 + w * v0
                        facc1 = facc1 + w * v1
                enc_v[coff + 2 * l, pl.ds(gof, 16)] = facc0
                enc_v[coff + 2 * l + 1, pl.ds(gof, 16)] = facc1
            return c3

        lax.fori_loop(0, _NG, acc_body, 0)

    def chunk_body(ch, carry):
        pltpu.sync_copy(coords.at[wid, ch], inp_v)

        def lvl(l, c2):
            b = l & 1

            @pl.when(l < _L)
            def _():
                idx_phase(l, b)
                fire(b, sems.at[b])

            @pl.when(l > 0)
            def _():
                drain(1 - b, sems.at[1 - b])
                acc_phase(l - 1, 1 - b)

            return c2

        lax.fori_loop(0, _L + 1, lvl, 0)
        pltpu.sync_copy(enc_v, out.at[:, pl.ds(wid * _PPT + ch * _CH, _CH)])
        return carry

    lax.fori_loop(0, _NCHUNK, chunk_body, 0)


_BN = 1024


def _mlp_body(enc_ref, w0_ref, w1_ref, w2_ref, out_ref):
    h = jnp.maximum(
        lax.dot_general(enc_ref[...], w0_ref[...], (((0,), (0,)), ((), ())),
                        preferred_element_type=jnp.float32,
                        precision=lax.Precision.HIGHEST), 0.0)
    h = jnp.maximum(
        jnp.dot(h, w1_ref[...], preferred_element_type=jnp.float32,
                precision=lax.Precision.HIGHEST), 0.0)
    out_ref[...] = jnp.dot(h, w2_ref[...], preferred_element_type=jnp.float32,
                           precision=lax.Precision.HIGHEST)


def _mlp(enc, W0, W1, W2):
    return pl.pallas_call(
        _mlp_body,
        grid=(_N // _BN,),
        in_specs=[
            pl.BlockSpec((_ENC, _BN), lambda i: (0, i)),
            pl.BlockSpec((_ENC, _HID), lambda i: (0, 0)),
            pl.BlockSpec((_HID, _HID), lambda i: (0, 0)),
            pl.BlockSpec((_HID, _ODIM), lambda i: (0, 0)),
        ],
        out_specs=pl.BlockSpec((_BN, _ODIM), lambda i: (i, 0)),
        out_shape=jax.ShapeDtypeStruct((_N, _ODIM), jnp.float32),
    )(enc, W0, W1, W2)


def kernel(face_centers, face_normals, pose_extended, pos_table, normal_table,
           pose_table, W0, W1, W2):
    coords = jnp.concatenate(
        [face_centers.T, face_normals.T, pose_extended.T], axis=0)
    coords = coords.reshape(10, _NW, _NCHUNK, _CH).transpose(1, 2, 0, 3)

    def native_view(tbl):
        return tbl.reshape(_L, _T // 128, 128, _F).transpose(0, 1, 3, 2).reshape(
            _TROWS, 16)

    pos_c, nrm_c, pose_c = _convert_tables(
        native_view(pos_table), native_view(normal_table),
        native_view(pose_table))
    enc = _encode_sc(coords, pos_c, nrm_c, pose_c, jnp.asarray(_RES_F32))
    return _mlp(enc, W0, W1, W2)


# speedup vs baseline: 224.4851x; 1.1157x over previous
import functools

import numpy as np
import jax
import jax.numpy as jnp
from jax import lax
from jax.experimental import pallas as pl
from jax.experimental.pallas import tpu as pltpu
from jax.experimental.pallas import tpu_sc as plsc

_L = 16
_F = 2
_T = 1 << 19
_MASK = _T - 1
_N = 131072
_HID = 64
_ODIM = 9
_ENC = 3 * _L * _F
_TROWS = _L * _T * _F // 16

_NC = 2
_NS = 16
_NW = _NC * _NS
_PPT = _N // _NW
_CH = 64
_NCHUNK = _PPT // _CH
_NG = _CH // 16
_NSLOT = 32
_NSTRM = _NSLOT // 2

_CBLK = 64
_BPT = _TROWS * 16 // 256 // _NW

_RES_F32 = np.floor(16.0 * (1.4472692012786865 ** np.arange(_L))).astype(np.float32)

_P1 = np.int32(np.int64(2654435761) - (1 << 32))
_P2 = np.int32(805459861)
_P3 = np.int32(np.int64(3674653429) - (1 << 32))

_mesh = plsc.VectorSubcoreMesh(core_axis_name="c", subcore_axis_name="s")
_sc_params = pltpu.CompilerParams(needs_layout_passes=False,
                                  use_tc_tiling_on_sc=False)

_tbl_sds = jax.ShapeDtypeStruct((_TROWS, 16), jnp.float32)


@functools.partial(
    pl.kernel,
    out_type=(_tbl_sds, _tbl_sds, _tbl_sds),
    mesh=_mesh,
    compiler_params=_sc_params,
    scratch_types=[
        pltpu.VMEM((2, _CBLK * 16, 16), jnp.float32),
        pltpu.VMEM((2, _CBLK * 16, 16), jnp.float32),
        pltpu.SemaphoreType.DMA((2,)),
        pltpu.SemaphoreType.DMA((2,)),
    ],
)
def _convert_tables(t0, t1, t2, o0, o1, o2, in_v, out_v, semi, semo):
    cid = lax.axis_index("c")
    sid = lax.axis_index("s")
    wid = sid * _NC + cid
    lane = lax.iota(jnp.int32, 16)
    half = lane >> 1
    parity = lane & 1
    p8 = parity * 8
    nch = _BPT // _CBLK
    cb16 = _CBLK * 16

    for tin, tout in ((t0, o0), (t1, o1), (t2, o2)):
        def chunk_body(c, carry):
            b = c & 1

            @pl.when(c < nch)
            def _():
                row0 = (wid * _BPT + c * _CBLK) * 16
                pltpu.async_copy(tin.at[pl.ds(row0, cb16)], in_v.at[b],
                                 semi.at[b])

            @pl.when(c > 0)
            def _():
                bb = 1 - b

                @pl.when(c > 2)
                def _():
                    pltpu.make_async_copy(
                        out_v.at[bb], tout.at[pl.ds(0, cb16)],
                        semo.at[bb]).wait()

                pltpu.make_async_copy(
                    tin.at[pl.ds(0, cb16)], in_v.at[bb], semi.at[bb]).wait()

                def blk_body(blk, c2):
                    for k in range(16):
                        src = plsc.load_gather(
                            in_v, [jnp.full((16,), bb, jnp.int32),
                                   blk * 16 + (k >> 1) + p8,
                                   8 * (k & 1) + half])
                        out_v[bb, blk * 16 + k, :] = src
                    return c2

                lax.fori_loop(0, _CBLK, blk_body, 0)
                row1 = (wid * _BPT + (c - 1) * _CBLK) * 16
                pltpu.async_copy(out_v.at[bb], tout.at[pl.ds(row1, cb16)],
                                 semo.at[bb])

            return carry

        lax.fori_loop(0, nch + 1, chunk_body, 0)
        pltpu.make_async_copy(out_v.at[0], tout.at[pl.ds(0, cb16)],
                              semo.at[0]).wait()
        pltpu.make_async_copy(out_v.at[1], tout.at[pl.ds(0, cb16)],
                              semo.at[1]).wait()


@functools.partial(
    pl.kernel,
    out_type=jax.ShapeDtypeStruct((_ENC, _N), jnp.float32),
    mesh=_mesh,
    compiler_params=_sc_params,
    scratch_types=[
        pltpu.VMEM((16,), jnp.float32),
        pltpu.VMEM((10, _CH), jnp.float32),
        pltpu.VMEM((2, 10, _CH), jnp.float32),
        pltpu.VMEM((2, _NSTRM, 2 * _CH), jnp.int32),
        pltpu.VMEM((2, _NSLOT, _CH), jnp.int32),
        pltpu.VMEM((2, _NSLOT * _CH, 16), jnp.float32),
        pltpu.VMEM((_ENC, _CH), jnp.float32),
        pltpu.SemaphoreType.DMA((2,)),
    ],
)
def _encode_sc(coords, pos_t, nrm_t, pose_t, res_in, out,
               res_v, inp_v, frac_v, idx_v, idxlo_v, rows_v, enc_v, sems):
    cid = lax.axis_index("c")
    sid = lax.axis_index("s")
    wid = sid * _NC + cid
    pltpu.sync_copy(res_in, res_v)

    lane = lax.iota(jnp.int32, 16)

    tables = ((0, 3, 0, 0), (3, 3, 8, 32), (6, 4, 16, 64))

    def idx_phase(l, buf):
        resb = plsc.load_gather(res_v, [jnp.full((16,), l, jnp.int32)])
        lofs = l * (_T * _F // 16)

        def idx_body(g, c3):
            gof = g * 16
            for roff, nd, soff, _coff in tables:
                h0 = []
                h1 = []
                for dd in range(nd):
                    x = inp_v[roff + dd, pl.ds(gof, 16)]
                    pos = x * resb
                    ip = pos.astype(jnp.int32)
                    frac_v[buf, roff + dd, pl.ds(gof, 16)] = pos - ip.astype(jnp.float32)
                    if dd == 0:
                        a0 = ip
                        a1 = ip + 1
                    else:
                        p = (_P1, _P2, _P3)[dd - 1]
                        a0 = ip * p
                        a1 = a0 + p
                    h0.append(a0)
                    h1.append(a1)
                for c in range(1 << nd):
                    h = None
                    for dd in range(nd):
                        term = h1[dd] if (c >> (nd - 1 - dd)) & 1 else h0[dd]
                        h = term if h is None else h ^ term
                    t = h & _MASK
                    s = soff + c
                    idx_v[buf, s >> 1, pl.ds((s & 1) * _CH + gof, 16)] = (t >> 3) + lofs
                    idxlo_v[buf, s, pl.ds(gof, 16)] = (t & 7) * 2
            return c3

        lax.fori_loop(0, _NG, idx_body, 0)

    def fire(buf, sem):
        for p in range(_NSTRM):
            tref = pos_t if p < 4 else (nrm_t if p < 8 else pose_t)
            pltpu.async_copy(
                tref.at[idx_v.at[buf, p]],
                rows_v.at[buf, pl.ds(p * 2 * _CH, 2 * _CH)],
                sem)

    def drain(buf, sem):
        pltpu.make_async_copy(
            pos_t.at[pl.ds(0, _NSLOT * _CH)],
            rows_v.at[buf],
            sem).wait()

    def acc_phase(l, buf):
        def acc_body(g, c3):
            gof = g * 16
            rowv = gof + lane
            for roff, nd, soff, coff in tables:
                fr = [frac_v[buf, roff + dd, pl.ds(gof, 16)] for dd in range(nd)]
                om = [1.0 - f for f in fr]
                p01 = [om[0] * om[1], om[0] * fr[1], fr[0] * om[1], fr[0] * fr[1]]
                if nd == 3:
                    tail = [om[2], fr[2]]
                else:
                    tail = [om[2] * om[3], om[2] * fr[3], fr[2] * om[3], fr[2] * fr[3]]
                facc0 = None
                facc1 = None
                for c in range(1 << nd):
                    w = p01[c >> (nd - 2)] * tail[c & ((1 << (nd - 2)) - 1)]
                    s = soff + c
                    col0 = idxlo_v[buf, s, pl.ds(gof, 16)]
                    v0 = plsc.load_gather(rows_v, [jnp.full((16,), buf, jnp.int32),
                                                   s * _CH + rowv, col0])
                    v1 = plsc.load_gather(rows_v, [jnp.full((16,), buf, jnp.int32),
                                                   s * _CH + rowv, col0 + 1])
                    if facc0 is None:
                        facc0 = w * v0
                        facc1 = w * v1
                    else:
                        facc0 = facc0 + w * v0
                        facc1 = facc1 + w * v1
                enc_v[coff + 2 * l, pl.ds(gof, 16)] = facc0
                enc_v[coff + 2 * l + 1, pl.ds(gof, 16)] = facc1
            return c3

        lax.fori_loop(0, _NG, acc_body, 0)

    def chunk_body(ch, carry):
        pltpu.sync_copy(coords.at[wid, ch], inp_v)

        def lvl(l, c2):
            b = l & 1

            @pl.when(l < _L)
            def _():
                idx_phase(l, b)
                fire(b, sems.at[b])

            @pl.when(l > 0)
            def _():
                drain(1 - b, sems.at[1 - b])
                acc_phase(l - 1, 1 - b)

            return c2

        lax.fori_loop(0, _L + 1, lvl, 0)
        pltpu.sync_copy(enc_v, out.at[:, pl.ds(wid * _PPT + ch * _CH, _CH)])
        return carry

    lax.fori_loop(0, _NCHUNK, chunk_body, 0)


_BN = 1024


def _mlp_body(enc_ref, w0_ref, w1_ref, w2_ref, out_ref):
    h = jnp.maximum(
        lax.dot_general(enc_ref[...], w0_ref[...], (((0,), (0,)), ((), ())),
                        preferred_element_type=jnp.float32), 0.0)
    h = jnp.maximum(
        jnp.dot(h, w1_ref[...], preferred_element_type=jnp.float32), 0.0)
    out_ref[...] = jnp.dot(h, w2_ref[...], preferred_element_type=jnp.float32)


def _mlp(enc, W0, W1, W2):
    return pl.pallas_call(
        _mlp_body,
        grid=(_N // _BN,),
        in_specs=[
            pl.BlockSpec((_ENC, _BN), lambda i: (0, i)),
            pl.BlockSpec((_ENC, _HID), lambda i: (0, 0)),
            pl.BlockSpec((_HID, _HID), lambda i: (0, 0)),
            pl.BlockSpec((_HID, _ODIM), lambda i: (0, 0)),
        ],
        out_specs=pl.BlockSpec((_BN, _ODIM), lambda i: (i, 0)),
        out_shape=jax.ShapeDtypeStruct((_N, _ODIM), jnp.float32),
    )(enc, W0, W1, W2)


def kernel(face_centers, face_normals, pose_extended, pos_table, normal_table,
           pose_table, W0, W1, W2):
    coords = jnp.concatenate(
        [face_centers.T, face_normals.T, pose_extended.T], axis=0)
    coords = coords.reshape(10, _NW, _NCHUNK, _CH).transpose(1, 2, 0, 3)

    def native_view(tbl):
        return tbl.reshape(_L, _T // 128, 128, _F).transpose(0, 1, 3, 2).reshape(
            _TROWS, 16)

    pos_c, nrm_c, pose_c = _convert_tables(
        native_view(pos_table), native_view(normal_table),
        native_view(pose_table))
    enc = _encode_sc(coords, pos_c, nrm_c, pose_c, jnp.asarray(_RES_F32))
    return _mlp(enc, W0, W1, W2)


# speedup vs baseline: 229.8107x; 1.0237x over previous
import functools

import numpy as np
import jax
import jax.numpy as jnp
from jax import lax
from jax.experimental import pallas as pl
from jax.experimental.pallas import tpu as pltpu
from jax.experimental.pallas import tpu_sc as plsc

_L = 16
_F = 2
_T = 1 << 19
_MASK = _T - 1
_N = 131072
_HID = 64
_ODIM = 9
_ENC = 3 * _L * _F
_TROWS = _L * _T * _F // 16

_NC = 2
_NS = 16
_NW = _NC * _NS
_PPT = _N // _NW
_CH = 64
_NCHUNK = _PPT // _CH
_NG = _CH // 16
_NSLOT = 32
_NSTRM = _NSLOT // 2

_CBLK = 64
_BPT = _TROWS * 16 // 256 // _NW

_RES_F32 = np.floor(16.0 * (1.4472692012786865 ** np.arange(_L))).astype(np.float32)

_P1 = np.int32(np.int64(2654435761) - (1 << 32))
_P2 = np.int32(805459861)
_P3 = np.int32(np.int64(3674653429) - (1 << 32))

_mesh = plsc.VectorSubcoreMesh(core_axis_name="c", subcore_axis_name="s")
_sc_params = pltpu.CompilerParams(needs_layout_passes=False,
                                  use_tc_tiling_on_sc=False)

_tbl_sds = jax.ShapeDtypeStruct((_TROWS, 16), jnp.float32)


@functools.partial(
    pl.kernel,
    out_type=(_tbl_sds, _tbl_sds, _tbl_sds),
    mesh=_mesh,
    compiler_params=_sc_params,
    scratch_types=[
        pltpu.VMEM((2, _CBLK * 16, 16), jnp.float32),
        pltpu.VMEM((2, _CBLK * 16, 16), jnp.float32),
        pltpu.SemaphoreType.DMA((2,)),
        pltpu.SemaphoreType.DMA((2,)),
    ],
)
def _convert_tables(t0, t1, t2, o0, o1, o2, in_v, out_v, semi, semo):
    cid = lax.axis_index("c")
    sid = lax.axis_index("s")
    wid = sid * _NC + cid
    lane = lax.iota(jnp.int32, 16)
    half = lane >> 1
    parity = lane & 1
    p8 = parity * 8
    nch = _BPT // _CBLK
    cb16 = _CBLK * 16

    for tin, tout in ((t0, o0), (t1, o1), (t2, o2)):
        def chunk_body(c, carry):
            b = c & 1

            @pl.when(c < nch)
            def _():
                row0 = (wid * _BPT + c * _CBLK) * 16
                pltpu.async_copy(tin.at[pl.ds(row0, cb16)], in_v.at[b],
                                 semi.at[b])

            @pl.when(c > 0)
            def _():
                bb = 1 - b

                @pl.when(c > 2)
                def _():
                    pltpu.make_async_copy(
                        out_v.at[bb], tout.at[pl.ds(0, cb16)],
                        semo.at[bb]).wait()

                pltpu.make_async_copy(
                    tin.at[pl.ds(0, cb16)], in_v.at[bb], semi.at[bb]).wait()

                def blk_body(blk, c2):
                    for k in range(16):
                        src = plsc.load_gather(
                            in_v, [jnp.full((16,), bb, jnp.int32),
                                   blk * 16 + (k >> 1) + p8,
                                   8 * (k & 1) + half])
                        out_v[bb, blk * 16 + k, :] = src
                    return c2

                lax.fori_loop(0, _CBLK, blk_body, 0)
                row1 = (wid * _BPT + (c - 1) * _CBLK) * 16
                pltpu.async_copy(out_v.at[bb], tout.at[pl.ds(row1, cb16)],
                                 semo.at[bb])

            return carry

        lax.fori_loop(0, nch + 1, chunk_body, 0)
        pltpu.make_async_copy(out_v.at[0], tout.at[pl.ds(0, cb16)],
                              semo.at[0]).wait()
        pltpu.make_async_copy(out_v.at[1], tout.at[pl.ds(0, cb16)],
                              semo.at[1]).wait()


@functools.partial(
    pl.kernel,
    out_type=jax.ShapeDtypeStruct((_ENC, _N), jnp.float32),
    mesh=_mesh,
    compiler_params=_sc_params,
    scratch_types=[
        pltpu.VMEM((16,), jnp.float32),
        pltpu.VMEM((10, _CH), jnp.float32),
        pltpu.VMEM((2, 10, _CH), jnp.float32),
        pltpu.VMEM((2, _NSTRM, 2 * _CH), jnp.int32),
        pltpu.VMEM((2, _NSLOT, _CH), jnp.int32),
        pltpu.VMEM((2, _NSLOT * _CH, 16), jnp.float32),
        pltpu.VMEM((_ENC, _CH), jnp.float32),
        pltpu.SemaphoreType.DMA((2,)),
    ],
)
def _encode_sc(coords, pos_t, nrm_t, pose_t, res_in, out,
               res_v, inp_v, frac_v, idx_v, idxlo_v, rows_v, enc_v, sems):
    cid = lax.axis_index("c")
    sid = lax.axis_index("s")
    wid = sid * _NC + cid
    pltpu.sync_copy(res_in, res_v)

    lane = lax.iota(jnp.int32, 16)

    tables = ((0, 3, 0, 0), (3, 3, 8, 32), (6, 4, 16, 64))

    def idx_phase(l, buf):
        resb = plsc.load_gather(res_v, [jnp.full((16,), l, jnp.int32)])
        lofs = l * (_T * _F // 16)

        def idx_body(g, c3):
            gof = g * 16
            for roff, nd, soff, _coff in tables:
                h0 = []
                h1 = []
                for dd in range(nd):
                    x = inp_v[roff + dd, pl.ds(gof, 16)]
                    pos = x * resb
                    ip = pos.astype(jnp.int32)
                    frac_v[buf, roff + dd, pl.ds(gof, 16)] = pos - ip.astype(jnp.float32)
                    if dd == 0:
                        a0 = ip
                        a1 = ip + 1
                    else:
                        p = (_P1, _P2, _P3)[dd - 1]
                        a0 = ip * p
                        a1 = a0 + p
                    h0.append(a0)
                    h1.append(a1)
                for c in range(1 << nd):
                    h = None
                    for dd in range(nd):
                        term = h1[dd] if (c >> (nd - 1 - dd)) & 1 else h0[dd]
                        h = term if h is None else h ^ term
                    t = h & _MASK
                    s = soff + c
                    idx_v[buf, s >> 1, pl.ds((s & 1) * _CH + gof, 16)] = (t >> 3) + lofs
                    idxlo_v[buf, s, pl.ds(gof, 16)] = (t & 7) * 2
            return c3

        lax.fori_loop(0, _NG, idx_body, 0)

    def fire(buf, sem):
        for p in range(_NSTRM):
            tref = pos_t if p < 4 else (nrm_t if p < 8 else pose_t)
            pltpu.async_copy(
                tref.at[idx_v.at[buf, p]],
                rows_v.at[buf, pl.ds(p * 2 * _CH, 2 * _CH)],
                sem)

    def drain(buf, sem):
        pltpu.make_async_copy(
            pos_t.at[pl.ds(0, _NSLOT * _CH)],
            rows_v.at[buf],
            sem).wait()

    def acc_phase(l, buf):
        def acc_body(g, c3):
            gof = g * 16
            rowv = gof + lane
            for roff, nd, soff, coff in tables:
                fr = [frac_v[buf, roff + dd, pl.ds(gof, 16)] for dd in range(nd)]
                om = [1.0 - f for f in fr]
                p01 = [om[0] * om[1], om[0] * fr[1], fr[0] * om[1], fr[0] * fr[1]]
                if nd == 3:
                    tail = [om[2], fr[2]]
                else:
                    tail = [om[2] * om[3], om[2] * fr[3], fr[2] * om[3], fr[2] * fr[3]]
                facc0 = None
                facc1 = None
                for c in range(1 << nd):
                    w = p01[c >> (nd - 2)] * tail[c & ((1 << (nd - 2)) - 1)]
                    s = soff + c
                    col0 = idxlo_v[buf, s, pl.ds(gof, 16)]
                    v0 = plsc.load_gather(rows_v, [jnp.full((16,), buf, jnp.int32),
                                                   s * _CH + rowv, col0])
                    v1 = plsc.load_gather(rows_v, [jnp.full((16,), buf, jnp.int32),
                                                   s * _CH + rowv, col0 + 1])
                    if facc0 is None:
                        facc0 = w * v0
                        facc1 = w * v1
                    else:
                        facc0 = facc0 + w * v0
                        facc1 = facc1 + w * v1
                enc_v[coff + 2 * l, pl.ds(gof, 16)] = facc0
                enc_v[coff + 2 * l + 1, pl.ds(gof, 16)] = facc1
            return c3

        lax.fori_loop(0, _NG, acc_body, 0)

    def chunk_body(ch, carry):
        pltpu.sync_copy(coords.at[wid, ch], inp_v)

        def lvl(l, c2):
            b = l & 1

            @pl.when(l < _L)
            def _():
                idx_phase(l, b)
                fire(b, sems.at[b])

            @pl.when(l > 0)
            def _():
                drain(1 - b, sems.at[1 - b])
                acc_phase(l - 1, 1 - b)

            return c2

        lax.fori_loop(0, _L + 1, lvl, 0)
        pltpu.sync_copy(enc_v, out.at[:, pl.ds(wid * _PPT + ch * _CH, _CH)])
        return carry

    lax.fori_loop(0, _NCHUNK, chunk_body, 0)


_BN = 4096


def _mlp_body(enc_ref, w0_ref, w1_ref, w2_ref, out_ref):
    h = jnp.maximum(
        lax.dot_general(enc_ref[...], w0_ref[...], (((0,), (0,)), ((), ())),
                        preferred_element_type=jnp.float32), 0.0)
    h = jnp.maximum(
        jnp.dot(h, w1_ref[...], preferred_element_type=jnp.float32), 0.0)
    out_ref[...] = jnp.dot(h, w2_ref[...], preferred_element_type=jnp.float32)


def _mlp(enc, W0, W1, W2):
    return pl.pallas_call(
        _mlp_body,
        grid=(_N // _BN,),
        in_specs=[
            pl.BlockSpec((_ENC, _BN), lambda i: (0, i)),
            pl.BlockSpec((_ENC, _HID), lambda i: (0, 0)),
            pl.BlockSpec((_HID, _HID), lambda i: (0, 0)),
            pl.BlockSpec((_HID, _ODIM), lambda i: (0, 0)),
        ],
        out_specs=pl.BlockSpec((_BN, _ODIM), lambda i: (i, 0)),
        out_shape=jax.ShapeDtypeStruct((_N, _ODIM), jnp.float32),
    )(enc, W0, W1, W2)


def kernel(face_centers, face_normals, pose_extended, pos_table, normal_table,
           pose_table, W0, W1, W2):
    coords = jnp.concatenate(
        [face_centers.T, face_normals.T, pose_extended.T], axis=0)
    coords = coords.reshape(10, _NW, _NCHUNK, _CH).transpose(1, 2, 0, 3)

    def native_view(tbl):
        return tbl.reshape(_L, _T // 128, 128, _F).transpose(0, 1, 3, 2).reshape(
            _TROWS, 16)

    pos_c, nrm_c, pose_c = _convert_tables(
        native_view(pos_table), native_view(normal_table),
        native_view(pose_table))
    enc = _encode_sc(coords, pos_c, nrm_c, pose_c, jnp.asarray(_RES_F32))
    return _mlp(enc, W0, W1, W2)
